# baseline (speedup 1.0000x reference)
import jax
import jax.numpy as jnp
from jax.experimental import pallas as pl
from jax.experimental.pallas import tpu as pltpu

_L = 2048
_D = 512
_G = 3 * _D
_PAD = 8
_C = 1024


def _localrnn_kernel(x_ref, wih_ref, whh_ref, bih_ref, bhh_ref, o_ref, g_s):
    bih = bih_ref[...]
    bhh = bhh_ref[...]
    wih = wih_ref[...]
    whh = whh_ref[...]

    def proj(c0):
        g_s[_PAD + c0:_PAD + c0 + _C, :] = (
            jnp.dot(x_ref[0, c0:c0 + _C, :], wih,
                    preferred_element_type=jnp.float32) + bih)

    g_s[0:_PAD, :] = jnp.broadcast_to(bih, (_PAD, _G))
    proj(0)

    bhh_r = bhh[:, 0:_D]
    bhh_z = bhh[:, _D:2 * _D]
    bhh_n = bhh[:, 2 * _D:]

    n_chunks = _L // _C
    for ci in range(n_chunks):
        c0 = ci * _C
        if ci + 1 < n_chunks:
            proj(c0 + _C)

        g0 = g_s[c0 + _PAD - 2:c0 + _PAD - 2 + _C, :]
        r = jax.nn.sigmoid(g0[:, 0:_D] + bhh_r)
        z = jax.nn.sigmoid(g0[:, _D:2 * _D] + bhh_z)
        n = jnp.tanh(g0[:, 2 * _D:] + r * bhh_n)
        h = (1.0 - z) * n

        for t in (1, 2):
            g = g_s[c0 + _PAD - 2 + t:c0 + _PAD - 2 + t + _C, :]
            gh = jnp.dot(h, whh, preferred_element_type=jnp.float32) + bhh
            r = jax.nn.sigmoid(g[:, 0:_D] + gh[:, 0:_D])
            z = jax.nn.sigmoid(g[:, _D:2 * _D] + gh[:, _D:2 * _D])
            n = jnp.tanh(g[:, 2 * _D:] + r * gh[:, 2 * _D:])
            h = (1.0 - z) * n + z * h

        o_ref[0, c0:c0 + _C, :] = h


@jax.jit
def kernel(x, W_ih, W_hh, b_ih, b_hh):
    B, L, D = x.shape
    wih_t = W_ih.T
    whh_t = W_hh.T
    bih2 = b_ih.reshape(1, _G)
    bhh2 = b_hh.reshape(1, _G)

    return pl.pallas_call(
        _localrnn_kernel,
        out_shape=jax.ShapeDtypeStruct((B, L, D), x.dtype),
        grid=(B,),
        in_specs=[
            pl.BlockSpec((1, L, D), lambda b: (b, 0, 0)),
            pl.BlockSpec((D, _G), lambda b: (0, 0)),
            pl.BlockSpec((D, _G), lambda b: (0, 0)),
            pl.BlockSpec((1, _G), lambda b: (0, 0)),
            pl.BlockSpec((1, _G), lambda b: (0, 0)),
        ],
        out_specs=pl.BlockSpec((1, L, D), lambda b: (b, 0, 0)),
        scratch_shapes=[pltpu.VMEM((_L + _PAD, _G), jnp.float32)],
        compiler_params=pltpu.CompilerParams(
            dimension_semantics=("parallel",),
            vmem_limit_bytes=56 * 1024 * 1024,
        ),
        name="localrnn_gru3",
    )(x, wih_t, whh_t, bih2, bhh2)


# speedup vs baseline: 1.1501x; 1.1501x over previous
import jax
import jax.numpy as jnp
from jax.experimental import pallas as pl
from jax.experimental.pallas import tpu as pltpu

_L = 2048
_D = 512
_G = 3 * _D
_PAD = 8
_C = 512


def _localrnn_kernel(x_ref, wih_ref, whh_ref, bih_ref, bhh_ref, o_ref, g_s):
    bih = bih_ref[...]
    bhh = bhh_ref[...]
    wih = wih_ref[...]
    whh = whh_ref[...]

    def proj(c0):
        g_s[_PAD + c0:_PAD + c0 + _C, :] = (
            jnp.dot(x_ref[0, c0:c0 + _C, :], wih,
                    preferred_element_type=jnp.float32) + bih)

    g_s[0:_PAD, :] = jnp.broadcast_to(bih, (_PAD, _G))
    proj(0)

    bhh_r = bhh[:, 0:_D]
    bhh_z = bhh[:, _D:2 * _D]
    bhh_n = bhh[:, 2 * _D:]

    _CE = _C + _PAD
    n_chunks = _L // _C
    for ci in range(n_chunks):
        c0 = ci * _C
        if ci + 1 < n_chunks:
            proj(c0 + _C)

        gE = g_s[c0:c0 + _CE, :]

        r = jax.nn.sigmoid(gE[:, 0:_D] + bhh_r)
        z = jax.nn.sigmoid(gE[:, _D:2 * _D] + bhh_z)
        n = jnp.tanh(gE[:, 2 * _D:] + r * bhh_n)
        h0 = (1.0 - z) * n

        h0s = jnp.concatenate([jnp.zeros((1, _D), jnp.float32), h0[:_CE - 1]],
                              axis=0)
        gh = jnp.dot(h0s, whh, preferred_element_type=jnp.float32) + bhh
        r = jax.nn.sigmoid(gE[:, 0:_D] + gh[:, 0:_D])
        z = jax.nn.sigmoid(gE[:, _D:2 * _D] + gh[:, _D:2 * _D])
        n = jnp.tanh(gE[:, 2 * _D:] + r * gh[:, 2 * _D:])
        h1 = (1.0 - z) * n + z * h0s

        h1s = h1[_PAD - 1:_CE - 1]
        g2 = gE[_PAD:, :]
        gh = jnp.dot(h1s, whh, preferred_element_type=jnp.float32) + bhh
        r = jax.nn.sigmoid(g2[:, 0:_D] + gh[:, 0:_D])
        z = jax.nn.sigmoid(g2[:, _D:2 * _D] + gh[:, _D:2 * _D])
        n = jnp.tanh(g2[:, 2 * _D:] + r * gh[:, 2 * _D:])
        h2 = (1.0 - z) * n + z * h1s

        o_ref[0, c0:c0 + _C, :] = h2


@jax.jit
def kernel(x, W_ih, W_hh, b_ih, b_hh):
    B, L, D = x.shape
    wih_t = W_ih.T
    whh_t = W_hh.T
    bih2 = b_ih.reshape(1, _G)
    bhh2 = b_hh.reshape(1, _G)

    return pl.pallas_call(
        _localrnn_kernel,
        out_shape=jax.ShapeDtypeStruct((B, L, D), x.dtype),
        grid=(B,),
        in_specs=[
            pl.BlockSpec((1, L, D), lambda b: (b, 0, 0)),
            pl.BlockSpec((D, _G), lambda b: (0, 0)),
            pl.BlockSpec((D, _G), lambda b: (0, 0)),
            pl.BlockSpec((1, _G), lambda b: (0, 0)),
            pl.BlockSpec((1, _G), lambda b: (0, 0)),
        ],
        out_specs=pl.BlockSpec((1, L, D), lambda b: (b, 0, 0)),
        scratch_shapes=[pltpu.VMEM((_L + _PAD, _G), jnp.float32)],
        compiler_params=pltpu.CompilerParams(
            dimension_semantics=("parallel",),
            vmem_limit_bytes=56 * 1024 * 1024,
        ),
        name="localrnn_gru3",
    )(x, wih_t, whh_t, bih2, bhh2)
